# baseline (speedup 1.0000x reference)
import functools

import jax
import jax.numpy as jnp
from jax import lax
from jax.experimental import pallas as pl
from jax.experimental.pallas import tpu as pltpu
from jax.experimental.pallas import tpu_sc as plsc

NC = 2
NS = 16
NW = NC * NS
CHUNK = 128
NBUF = 6


@functools.partial(jax.jit, static_argnames=("n_chunks", "d_embed"))
def _embed_sc(x2d, w, n_chunks, d_embed):
    n_total = NW * n_chunks * CHUNK
    b_per_w = n_chunks * CHUNK
    mesh = plsc.VectorSubcoreMesh(core_axis_name="c", subcore_axis_name="s")

    def body(x_hbm, w_hbm, out_hbm, idx_v, rows_v, gsems, wsems):
        wid = lax.axis_index("s") * NC + lax.axis_index("c")
        base = wid * b_per_w
        pltpu.sync_copy(x_hbm.at[pl.ds(wid * n_chunks, n_chunks)], idx_v)

        def idx_at(j):
            return idx_v.at[j]

        def out_at(j):
            return out_hbm.at[pl.ds(base + j * CHUNK, CHUNK)]

        AHEAD = 2
        for j in range(min(AHEAD, n_chunks)):
            pltpu.async_copy(w_hbm.at[idx_at(j)], rows_v.at[j], gsems.at[j])
        for j in range(n_chunks):
            b = j % NBUF
            pltpu.make_async_copy(
                w_hbm.at[idx_at(j)], rows_v.at[b], gsems.at[b]).wait()
            pltpu.async_copy(rows_v.at[b], out_at(j), wsems.at[b])
            nj = j + AHEAD
            if nj < n_chunks:
                bb = nj % NBUF
                if nj >= NBUF:
                    pltpu.make_async_copy(rows_v.at[bb], out_at(nj - NBUF),
                                          wsems.at[bb]).wait()
                pltpu.async_copy(w_hbm.at[idx_at(nj)], rows_v.at[bb],
                                 gsems.at[bb])
        for j in range(max(0, n_chunks - NBUF), n_chunks):
            b = j % NBUF
            pltpu.make_async_copy(rows_v.at[b], out_at(j), wsems.at[b]).wait()

    run = pl.kernel(
        body,
        out_type=jax.ShapeDtypeStruct((n_total, d_embed), w.dtype),
        mesh=mesh,
        scratch_types=[
            pltpu.VMEM((n_chunks, CHUNK), jnp.int32),
            pltpu.VMEM((NBUF, CHUNK, d_embed), w.dtype),
            pltpu.SemaphoreType.DMA((NBUF,)),
            pltpu.SemaphoreType.DMA((NBUF,)),
        ],
    )
    return run(x2d, w)


def kernel(x, W_E):
    n_total = x.size
    d_embed = W_E.shape[1]
    assert n_total % (NW * CHUNK) == 0
    n_chunks = n_total // (NW * CHUNK)
    x2d = x.reshape(n_total // CHUNK, CHUNK).astype(jnp.int32)
    out = _embed_sc(x2d, W_E, n_chunks, d_embed)
    return out.reshape(x.shape + (d_embed,))


# speedup vs baseline: 1.0269x; 1.0269x over previous
import functools

import jax
import jax.numpy as jnp
from jax import lax
from jax.experimental import pallas as pl
from jax.experimental.pallas import tpu as pltpu
from jax.experimental.pallas import tpu_sc as plsc

NC = 2
NS = 16
NW = NC * NS
CHUNK = 128
NBUF = 7


@functools.partial(jax.jit, static_argnames=("n_chunks", "d_embed"))
def _embed_sc(x2d, w, n_chunks, d_embed):
    n_total = NW * n_chunks * CHUNK
    b_per_w = n_chunks * CHUNK
    mesh = plsc.VectorSubcoreMesh(core_axis_name="c", subcore_axis_name="s")

    def body(x_hbm, w_hbm, out_hbm, idx_v, rows_v, gsems, wsems):
        wid = lax.axis_index("s") * NC + lax.axis_index("c")
        base = wid * b_per_w
        pltpu.sync_copy(x_hbm.at[pl.ds(wid * n_chunks, n_chunks)], idx_v)

        def idx_at(j):
            return idx_v.at[j]

        def out_at(j):
            return out_hbm.at[pl.ds(base + j * CHUNK, CHUNK)]

        AHEAD = NBUF
        for j in range(min(AHEAD, n_chunks)):
            pltpu.async_copy(w_hbm.at[idx_at(j)], rows_v.at[j], gsems.at[j])
        for j in range(n_chunks):
            b = j % NBUF
            pltpu.make_async_copy(
                w_hbm.at[idx_at(j)], rows_v.at[b], gsems.at[b]).wait()
            pltpu.async_copy(rows_v.at[b], out_at(j), wsems.at[b])
            nj = j + AHEAD
            if nj < n_chunks:
                bb = nj % NBUF
                if nj >= NBUF:
                    pltpu.make_async_copy(rows_v.at[bb], out_at(nj - NBUF),
                                          wsems.at[bb]).wait()
                pltpu.async_copy(w_hbm.at[idx_at(nj)], rows_v.at[bb],
                                 gsems.at[bb])
        for j in range(max(0, n_chunks - NBUF), n_chunks):
            b = j % NBUF
            pltpu.make_async_copy(rows_v.at[b], out_at(j), wsems.at[b]).wait()

    run = pl.kernel(
        body,
        out_type=jax.ShapeDtypeStruct((n_total, d_embed), w.dtype),
        mesh=mesh,
        scratch_types=[
            pltpu.VMEM((n_chunks, CHUNK), jnp.int32),
            pltpu.VMEM((NBUF, CHUNK, d_embed), w.dtype),
            pltpu.SemaphoreType.DMA((NBUF,)),
            pltpu.SemaphoreType.DMA((NBUF,)),
        ],
    )
    return run(x2d, w)


def kernel(x, W_E):
    n_total = x.size
    d_embed = W_E.shape[1]
    assert n_total % (NW * CHUNK) == 0
    n_chunks = n_total // (NW * CHUNK)
    x2d = x.reshape(n_total // CHUNK, CHUNK).astype(jnp.int32)
    out = _embed_sc(x2d, W_E, n_chunks, d_embed)
    return out.reshape(x.shape + (d_embed,))


# speedup vs baseline: 1.0491x; 1.0216x over previous
import functools

import jax
import jax.numpy as jnp
from jax import lax
from jax.experimental import pallas as pl
from jax.experimental.pallas import tpu as pltpu
from jax.experimental.pallas import tpu_sc as plsc

NC = 2
NS = 16
NW = NC * NS
CHUNK = 128
NBUF = 6


@functools.partial(jax.jit, static_argnames=("n_chunks", "d_embed"))
def _embed_sc(x2d, w, n_chunks, d_embed):
    n_total = NW * n_chunks * CHUNK
    b_per_w = n_chunks * CHUNK
    mesh = plsc.VectorSubcoreMesh(core_axis_name="c", subcore_axis_name="s")

    def body(x_hbm, w_hbm, out_hbm, idx_v, rows_v, gsems, wsems):
        wid = lax.axis_index("s") * NC + lax.axis_index("c")
        base = wid * b_per_w
        pltpu.sync_copy(x_hbm.at[pl.ds(wid * n_chunks, n_chunks)], idx_v)

        def idx_at(j):
            return idx_v.at[j]

        def out_at(j):
            return out_hbm.at[pl.ds(base + j * CHUNK, CHUNK)]

        AHEAD = NBUF
        for j in range(min(AHEAD, n_chunks)):
            pltpu.async_copy(w_hbm.at[idx_at(j)], rows_v.at[j], gsems.at[j])
        for j in range(n_chunks):
            b = j % NBUF
            pltpu.make_async_copy(
                w_hbm.at[idx_at(j)], rows_v.at[b], gsems.at[b]).wait()
            pltpu.async_copy(rows_v.at[b], out_at(j), wsems.at[b])
            nj = j + AHEAD
            if nj < n_chunks:
                bb = nj % NBUF
                if nj >= NBUF:
                    pltpu.make_async_copy(rows_v.at[bb], out_at(nj - NBUF),
                                          wsems.at[bb]).wait()
                pltpu.async_copy(w_hbm.at[idx_at(nj)], rows_v.at[bb],
                                 gsems.at[bb])
        for j in range(max(0, n_chunks - NBUF), n_chunks):
            b = j % NBUF
            pltpu.make_async_copy(rows_v.at[b], out_at(j), wsems.at[b]).wait()

    run = pl.kernel(
        body,
        out_type=jax.ShapeDtypeStruct((n_total, d_embed), w.dtype),
        mesh=mesh,
        scratch_types=[
            pltpu.VMEM((n_chunks, CHUNK), jnp.int32),
            pltpu.VMEM((NBUF, CHUNK, d_embed), w.dtype),
            pltpu.SemaphoreType.DMA((NBUF,)),
            pltpu.SemaphoreType.DMA((NBUF,)),
        ],
    )
    return run(x2d, w)


def kernel(x, W_E):
    n_total = x.size
    d_embed = W_E.shape[1]
    assert n_total % (NW * CHUNK) == 0
    n_chunks = n_total // (NW * CHUNK)
    x2d = x.reshape(n_total // CHUNK, CHUNK).astype(jnp.int32)
    out = _embed_sc(x2d, W_E, n_chunks, d_embed)
    return out.reshape(x.shape + (d_embed,))


# speedup vs baseline: 1.0678x; 1.0178x over previous
import functools

import jax
import jax.numpy as jnp
from jax import lax
from jax.experimental import pallas as pl
from jax.experimental.pallas import tpu as pltpu
from jax.experimental.pallas import tpu_sc as plsc

NC = 2
NS = 16
NW = NC * NS
CHUNK = 128
NBUF = 6


@functools.partial(jax.jit, static_argnames=("n_chunks", "d_embed"))
def _embed_sc(x, w, n_chunks, d_embed):
    n_total = NW * n_chunks * CHUNK
    b_per_w = n_chunks * CHUNK
    seq = x.shape[-1]
    mesh = plsc.VectorSubcoreMesh(core_axis_name="c", subcore_axis_name="s")

    def body(x_hbm, w_hbm, out_hbm, idx1_v, idx_v, rows_v, gsems, wsems):
        wid = lax.axis_index("s") * NC + lax.axis_index("c")
        base = wid * b_per_w
        row = base // seq
        col = base % seq
        pltpu.sync_copy(x_hbm.at[row, pl.ds(col, b_per_w)], idx1_v)
        for k in range(b_per_w // 16):
            idx_v[k * 16 // CHUNK, pl.ds((k * 16) % CHUNK, 16)] = (
                idx1_v[pl.ds(k * 16, 16)])

        def idx_at(j):
            return idx_v.at[j]

        def out_at(j):
            return out_hbm.at[pl.ds(base + j * CHUNK, CHUNK)]

        AHEAD = NBUF
        for j in range(min(AHEAD, n_chunks)):
            pltpu.async_copy(w_hbm.at[idx_at(j)], rows_v.at[j], gsems.at[j])
        for j in range(n_chunks):
            b = j % NBUF
            pltpu.make_async_copy(
                w_hbm.at[idx_at(j)], rows_v.at[b], gsems.at[b]).wait()
            pltpu.async_copy(rows_v.at[b], out_at(j), wsems.at[b])
            nj = j + AHEAD
            if nj < n_chunks:
                bb = nj % NBUF
                if nj >= NBUF:
                    pltpu.make_async_copy(rows_v.at[bb], out_at(nj - NBUF),
                                          wsems.at[bb]).wait()
                pltpu.async_copy(w_hbm.at[idx_at(nj)], rows_v.at[bb],
                                 gsems.at[bb])
        for j in range(max(0, n_chunks - NBUF), n_chunks):
            b = j % NBUF
            pltpu.make_async_copy(rows_v.at[b], out_at(j), wsems.at[b]).wait()

    run = pl.kernel(
        body,
        out_type=jax.ShapeDtypeStruct((n_total, d_embed), w.dtype),
        mesh=mesh,
        scratch_types=[
            pltpu.VMEM((b_per_w,), jnp.int32),
            pltpu.VMEM((n_chunks, CHUNK), jnp.int32),
            pltpu.VMEM((NBUF, CHUNK, d_embed), w.dtype),
            pltpu.SemaphoreType.DMA((NBUF,)),
            pltpu.SemaphoreType.DMA((NBUF,)),
        ],
    )
    return run(x, w)


def kernel(x, W_E):
    n_total = x.size
    d_embed = W_E.shape[1]
    assert n_total % (NW * CHUNK) == 0
    b_per_w = n_total // NW
    assert x.shape[-1] % b_per_w == 0
    n_chunks = b_per_w // CHUNK
    out = _embed_sc(x.astype(jnp.int32), W_E, n_chunks, d_embed)
    return out.reshape(x.shape + (d_embed,))
